# Initial kernel scaffold; baseline (speedup 1.0000x reference)
import jax
import jax.numpy as jnp
from jax.experimental import pallas as pl


def kernel(input_ids, base_weight, new_weight):
    raise NotImplementedError("write your pallas kernel here")


# speedup vs baseline: 13.1591x; 13.1591x over previous
import functools

import jax
import jax.numpy as jnp
from jax import lax
from jax.experimental import pallas as pl
from jax.experimental.pallas import tpu as pltpu
from jax.experimental.pallas import tpu_sc as plsc

_NC = 2
_NS = 16
_NW = _NC * _NS
_W = 256


def _gather_sc(table, idx, n, d):
    b_per_w = n // _NW
    n_chunks = b_per_w // _W
    mesh = plsc.VectorSubcoreMesh(core_axis_name="c", subcore_axis_name="s")

    @functools.partial(
        pl.kernel,
        out_type=jax.ShapeDtypeStruct((n, d), jnp.float32),
        mesh=mesh,
        scratch_types=[
            pltpu.VMEM((_W,), jnp.int32),
            pltpu.VMEM((_W, d), jnp.float32),
            pltpu.SemaphoreType.DMA,
        ],
    )
    def gather_kernel(table_hbm, idx_hbm, out_hbm, idx_v, rows_v, sem):
        wid = lax.axis_index("s") * _NC + lax.axis_index("c")
        base = wid * b_per_w

        @pl.loop(0, n_chunks)
        def _(i):
            off = base + i * _W
            pltpu.sync_copy(idx_hbm.at[pl.ds(off, _W)], idx_v)
            pltpu.async_copy(table_hbm.at[idx_v], rows_v, sem).wait()
            pltpu.sync_copy(rows_v, out_hbm.at[pl.ds(off, _W)])

    return gather_kernel(table, idx)


def kernel(input_ids, base_weight, new_weight):
    b, h = input_ids.shape
    d = base_weight.shape[1]
    table = jnp.concatenate([base_weight, new_weight], axis=0)
    idx = input_ids.reshape(-1).astype(jnp.int32)
    out = _gather_sc(table, idx, idx.shape[0], d)
    return out.reshape(b, h, d)


# speedup vs baseline: 17.3084x; 1.3153x over previous
import functools

import jax
import jax.numpy as jnp
from jax import lax
from jax.experimental import pallas as pl
from jax.experimental.pallas import tpu as pltpu
from jax.experimental.pallas import tpu_sc as plsc

_NC = 2
_NS = 16
_NW = _NC * _NS
_W = 256


def _gather_sc(table, idx, n, d):
    b_per_w = n // _NW
    n_chunks = b_per_w // _W
    mesh = plsc.VectorSubcoreMesh(core_axis_name="c", subcore_axis_name="s")

    @functools.partial(
        pl.kernel,
        out_type=jax.ShapeDtypeStruct((n, d), jnp.float32),
        mesh=mesh,
        scratch_types=[
            pltpu.VMEM((b_per_w,), jnp.int32),
            pltpu.VMEM((_W, d), jnp.float32),
            pltpu.VMEM((_W, d), jnp.float32),
            pltpu.SemaphoreType.DMA,
            pltpu.SemaphoreType.DMA,
            pltpu.SemaphoreType.DMA,
        ],
    )
    def gather_kernel(table_hbm, idx_hbm, out_hbm, idx_v, rows0, rows1,
                      gsem, osem0, osem1):
        wid = lax.axis_index("s") * _NC + lax.axis_index("c")
        base = wid * b_per_w
        pltpu.sync_copy(idx_hbm.at[pl.ds(base, b_per_w)], idx_v)

        rows = (rows0, rows1)
        osem = (osem0, osem1)

        def gather_chunk(c, buf):
            pltpu.async_copy(
                table_hbm.at[idx_v.at[pl.ds(c * _W, _W)]], rows[buf], gsem
            ).wait()

        def put_chunk(c, buf):
            pltpu.async_copy(rows[buf], out_hbm.at[pl.ds(base + c * _W, _W)],
                             osem[buf])

        gather_chunk(0, 0)
        put_chunk(0, 0)

        @pl.loop(1, n_chunks - 1, step=2)
        def _(c):
            gather_chunk(c, 1)
            put_chunk(c, 1)
            pltpu.make_async_copy(rows[0], out_hbm.at[pl.ds(base, _W)],
                                  osem[0]).wait()
            gather_chunk(c + 1, 0)
            put_chunk(c + 1, 0)
            pltpu.make_async_copy(rows[1], out_hbm.at[pl.ds(base, _W)],
                                  osem[1]).wait()

        gather_chunk(n_chunks - 1, 1)
        put_chunk(n_chunks - 1, 1)
        pltpu.make_async_copy(rows[1], out_hbm.at[pl.ds(base, _W)],
                              osem[1]).wait()
        pltpu.make_async_copy(rows[0], out_hbm.at[pl.ds(base, _W)],
                              osem[0]).wait()

    return gather_kernel(table, idx)


def kernel(input_ids, base_weight, new_weight):
    b, h = input_ids.shape
    d = base_weight.shape[1]
    table = jnp.concatenate([base_weight, new_weight], axis=0)
    idx = input_ids.reshape(-1).astype(jnp.int32)
    out = _gather_sc(table, idx, idx.shape[0], d)
    return out.reshape(b, h, d)


# speedup vs baseline: 17.4364x; 1.0074x over previous
import functools

import jax
import jax.numpy as jnp
from jax import lax
from jax.experimental import pallas as pl
from jax.experimental.pallas import tpu as pltpu
from jax.experimental.pallas import tpu_sc as plsc

_NC = 2
_NS = 16
_NW = _NC * _NS
_W = 256


def _gather_sc(table, idx, n, d):
    b_per_w = n // _NW
    n_chunks = b_per_w // _W
    mesh = plsc.VectorSubcoreMesh(core_axis_name="c", subcore_axis_name="s")

    assert n_chunks % 3 == 1 and n_chunks >= 4

    @functools.partial(
        pl.kernel,
        out_type=jax.ShapeDtypeStruct((n, d), jnp.float32),
        mesh=mesh,
        scratch_types=[
            pltpu.VMEM((b_per_w,), jnp.int32),
            pltpu.VMEM((_W, d), jnp.float32),
            pltpu.VMEM((_W, d), jnp.float32),
            pltpu.VMEM((_W, d), jnp.float32),
            pltpu.SemaphoreType.DMA,
            pltpu.SemaphoreType.DMA,
            pltpu.SemaphoreType.DMA,
            pltpu.SemaphoreType.DMA,
            pltpu.SemaphoreType.DMA,
            pltpu.SemaphoreType.DMA,
        ],
    )
    def gather_kernel(table_hbm, idx_hbm, out_hbm, idx_v, rows0, rows1, rows2,
                      gsem0, gsem1, gsem2, osem0, osem1, osem2):
        wid = lax.axis_index("s") * _NC + lax.axis_index("c")
        base = wid * b_per_w
        pltpu.sync_copy(idx_hbm.at[pl.ds(base, b_per_w)], idx_v)

        rows = (rows0, rows1, rows2)
        gsem = (gsem0, gsem1, gsem2)
        osem = (osem0, osem1, osem2)

        def gstart(c, b):
            pltpu.async_copy(
                table_hbm.at[idx_v.at[pl.ds(c * _W, _W)]], rows[b], gsem[b]
            )

        def gwait(b):
            pltpu.make_async_copy(
                table_hbm.at[idx_v.at[pl.ds(0, _W)]], rows[b], gsem[b]
            ).wait()

        def ostart(c, b):
            pltpu.async_copy(rows[b], out_hbm.at[pl.ds(base + c * _W, _W)],
                             osem[b])

        def owait(b):
            pltpu.make_async_copy(rows[b], out_hbm.at[pl.ds(base, _W)],
                                  osem[b]).wait()

        gstart(0, 0)
        gstart(1, 1)
        gstart(2, 2)
        gwait(0)
        ostart(0, 0)

        @pl.loop(1, n_chunks - 3, step=3)
        def _(c):
            for j in range(3):
                b = (1 + j) % 3
                pb = (b + 2) % 3
                owait(pb)
                gstart(c + j + 2, pb)
                gwait(b)
                ostart(c + j, b)

        owait(0)
        gstart(n_chunks - 1, 0)
        gwait(1)
        ostart(n_chunks - 3, 1)
        gwait(2)
        ostart(n_chunks - 2, 2)
        gwait(0)
        ostart(n_chunks - 1, 0)
        owait(1)
        owait(2)
        owait(0)

    return gather_kernel(table, idx)


def kernel(input_ids, base_weight, new_weight):
    b, h = input_ids.shape
    d = base_weight.shape[1]
    table = jnp.concatenate([base_weight, new_weight], axis=0)
    idx = input_ids.reshape(-1).astype(jnp.int32)
    out = _gather_sc(table, idx, idx.shape[0], d)
    return out.reshape(b, h, d)
